# baseline (speedup 1.0000x reference)
import functools

import jax
import jax.numpy as jnp
from jax import lax
from jax.experimental import pallas as pl
from jax.experimental.pallas import tpu as pltpu
from jax.experimental.pallas import tpu_sc as plsc

BATCH = 32
POS = 1024
DIM = 768

NUM_CORES = 2
NUM_SUBCORES = 16
NW = NUM_CORES * NUM_SUBCORES
PPW = POS // NW
CHUNK = PPW * DIM
LANES = 16
NVEC = CHUNK // LANES

_MESH = plsc.VectorSubcoreMesh(
    core_axis_name="c", subcore_axis_name="s",
    num_cores=NUM_CORES, num_subcores=NUM_SUBCORES)


@functools.partial(
    pl.kernel,
    out_type=jax.ShapeDtypeStruct((BATCH, NW, CHUNK), jnp.float32),
    mesh=_MESH,
    scratch_types=[
        pltpu.VMEM((CHUNK,), jnp.float32),
        pltpu.VMEM((CHUNK,), jnp.float32),
        pltpu.VMEM((CHUNK,), jnp.float32),
        pltpu.VMEM((CHUNK,), jnp.float32),
        pltpu.VMEM((CHUNK,), jnp.float32),
        pltpu.SemaphoreType.DMA,
        pltpu.SemaphoreType.DMA,
        pltpu.SemaphoreType.DMA,
        pltpu.SemaphoreType.DMA,
    ],
)
def _pos_add_sc(x_hbm, pos_hbm, out_hbm, pos_v, in0, in1, ob0, ob1,
                si0, si1, so0, so1):
    w = lax.axis_index("s") * NUM_CORES + lax.axis_index("c")
    inb = (in0, in1)
    outb = (ob0, ob1)
    sin = (si0, si1)
    sout = (so0, so1)

    pltpu.async_copy(x_hbm.at[0, w], in0, si0)
    pltpu.async_copy(x_hbm.at[1, w], in1, si1)
    pltpu.sync_copy(pos_hbm.at[w], pos_v)

    for b in range(BATCH):
        k = b & 1
        pltpu.make_async_copy(x_hbm.at[b, w], inb[k], sin[k]).wait()
        if b >= 2:
            pltpu.make_async_copy(outb[k], out_hbm.at[b - 2, w],
                                  sout[k]).wait()

        def add_body(i, carry, ib=inb[k], ob=outb[k]):
            sl = pl.ds(i * LANES, LANES)
            ob[sl] = ib[sl] + pos_v[sl]
            return carry

        lax.fori_loop(0, NVEC, add_body, 0, unroll=8)

        pltpu.async_copy(outb[k], out_hbm.at[b, w], sout[k])
        if b + 2 < BATCH:
            pltpu.async_copy(x_hbm.at[b + 2, w], inb[k], sin[k])

    pltpu.make_async_copy(ob0, out_hbm.at[BATCH - 2, w], so0).wait()
    pltpu.make_async_copy(ob1, out_hbm.at[BATCH - 1, w], so1).wait()


def kernel(inputs, pos_table):
    x = inputs.reshape(BATCH, NW, CHUNK)
    p = pos_table.reshape(NW, CHUNK)
    out = _pos_add_sc(x, p)
    return out.reshape(BATCH, POS, DIM)


# speedup vs baseline: 1.6697x; 1.6697x over previous
import functools

import jax
import jax.numpy as jnp
from jax import lax
from jax.experimental import pallas as pl
from jax.experimental.pallas import tpu as pltpu
from jax.experimental.pallas import tpu_sc as plsc

BATCH = 32
POS = 1024
DIM = 768

NUM_CORES = 2
NUM_SUBCORES = 16
NW = NUM_CORES * NUM_SUBCORES
PPW = POS // NW
CHUNK = PPW * DIM
LANES = 16
NVEC = CHUNK // LANES

_MESH = plsc.VectorSubcoreMesh(
    core_axis_name="c", subcore_axis_name="s",
    num_cores=NUM_CORES, num_subcores=NUM_SUBCORES)


@functools.partial(
    pl.kernel,
    out_type=jax.ShapeDtypeStruct((BATCH, NW, CHUNK), jnp.float32),
    mesh=_MESH,
    scratch_types=[
        pltpu.VMEM((CHUNK,), jnp.float32),
        pltpu.VMEM((CHUNK,), jnp.float32),
        pltpu.VMEM((CHUNK,), jnp.float32),
        pltpu.VMEM((CHUNK,), jnp.float32),
        pltpu.VMEM((CHUNK,), jnp.float32),
        pltpu.SemaphoreType.DMA,
        pltpu.SemaphoreType.DMA,
        pltpu.SemaphoreType.DMA,
        pltpu.SemaphoreType.DMA,
    ],
)
def _pos_add_sc(x_hbm, pos_hbm, out_hbm, pos_v, in0, in1, ob0, ob1,
                si0, si1, so0, so1):
    w = lax.axis_index("s") * NUM_CORES + lax.axis_index("c")
    inb = (in0, in1)
    outb = (ob0, ob1)
    sin = (si0, si1)
    sout = (so0, so1)

    pltpu.async_copy(x_hbm.at[0, w], in0, si0)
    pltpu.async_copy(x_hbm.at[1, w], in1, si1)
    pltpu.sync_copy(pos_hbm.at[w], pos_v)

    for b in range(BATCH):
        k = b & 1
        pltpu.make_async_copy(x_hbm.at[b, w], inb[k], sin[k]).wait()
        if b >= 2:
            pltpu.make_async_copy(outb[k], out_hbm.at[b - 2, w],
                                  sout[k]).wait()

        ib, ob = inb[k], outb[k]

        @plsc.parallel_loop(0, CHUNK, step=LANES, unroll=8)
        def _(i, ib=ib, ob=ob):
            sl = pl.ds(i, LANES)
            ob[sl] = ib[sl] + pos_v[sl]

        pltpu.async_copy(outb[k], out_hbm.at[b, w], sout[k])
        if b + 2 < BATCH:
            pltpu.async_copy(x_hbm.at[b + 2, w], inb[k], sin[k])

    pltpu.make_async_copy(ob0, out_hbm.at[BATCH - 2, w], so0).wait()
    pltpu.make_async_copy(ob1, out_hbm.at[BATCH - 1, w], so1).wait()


def kernel(inputs, pos_table):
    x = inputs.reshape(BATCH, NW, CHUNK)
    p = pos_table.reshape(NW, CHUNK)
    out = _pos_add_sc(x, p)
    return out.reshape(BATCH, POS, DIM)


# speedup vs baseline: 5.0641x; 3.0329x over previous
import functools

import jax
import jax.numpy as jnp
from jax import lax
from jax.experimental import pallas as pl
from jax.experimental.pallas import tpu as pltpu
from jax.experimental.pallas import tpu_sc as plsc

BATCH = 32
POS = 1024
DIM = 768

NUM_CORES = 2
NUM_SUBCORES = 16
NW = NUM_CORES * NUM_SUBCORES
PPW = POS // NW
LANES = 16
NPAIR = BATCH // 2

_MESH = plsc.VectorSubcoreMesh(
    core_axis_name="c", subcore_axis_name="s",
    num_cores=NUM_CORES, num_subcores=NUM_SUBCORES)


@functools.partial(
    pl.kernel,
    out_type=jax.ShapeDtypeStruct((BATCH, POS, DIM), jnp.float32),
    mesh=_MESH,
    scratch_types=[
        pltpu.VMEM((PPW, DIM), jnp.float32),
        pltpu.VMEM((PPW, DIM), jnp.float32),
        pltpu.VMEM((PPW, DIM), jnp.float32),
        pltpu.VMEM((PPW, DIM), jnp.float32),
        pltpu.VMEM((PPW, DIM), jnp.float32),
        pltpu.SemaphoreType.DMA,
        pltpu.SemaphoreType.DMA,
        pltpu.SemaphoreType.DMA,
        pltpu.SemaphoreType.DMA,
    ],
)
def _pos_add_sc(x_hbm, pos_hbm, out_hbm, pos_v, in0, in1, ob0, ob1,
                si0, si1, so0, so1):
    w = lax.axis_index("s") * NUM_CORES + lax.axis_index("c")
    r0 = w * PPW
    rows = pl.ds(r0, PPW)

    pltpu.async_copy(x_hbm.at[0, rows], in0, si0)
    pltpu.async_copy(x_hbm.at[1, rows], in1, si1)
    pltpu.sync_copy(pos_hbm.at[rows], pos_v)

    slots = ((in0, ob0, si0, so0), (in1, ob1, si1, so1))

    def pair_body(j, carry):
        for k in range(2):
            ib, ob, si, so = slots[k]
            b = 2 * j + k
            pltpu.make_async_copy(x_hbm.at[b, rows], ib, si).wait()

            @pl.when(j >= 1)
            def _wait_out(ob=ob, so=so, b=b):
                pltpu.make_async_copy(ob, out_hbm.at[b - 2, rows], so).wait()

            @plsc.parallel_loop(0, PPW)
            def _add(i, ib=ib, ob=ob):
                for c in range(0, DIM, LANES):
                    sl = pl.ds(c, LANES)
                    ob[i, sl] = ib[i, sl] + pos_v[i, sl]

            pltpu.async_copy(ob, out_hbm.at[b, rows], so)

            @pl.when(j < NPAIR - 1)
            def _prefetch(ib=ib, si=si, b=b):
                pltpu.async_copy(x_hbm.at[b + 2, rows], ib, si)
        return carry

    lax.fori_loop(0, NPAIR, pair_body, 0)

    pltpu.make_async_copy(ob0, out_hbm.at[BATCH - 2, rows], so0).wait()
    pltpu.make_async_copy(ob1, out_hbm.at[BATCH - 1, rows], so1).wait()


def kernel(inputs, pos_table):
    return _pos_add_sc(inputs, pos_table)
